# baseline (speedup 1.0000x reference)
import math

import jax
import jax.numpy as jnp
from jax.experimental import pallas as pl
from jax.experimental.pallas import tpu as pltpu

V = 9
EDGE_DIM = 5
HID_NODE = 13
NODE_DIM = 16
HIDDEN = 32
HEADS = 4
HH = HEADS * HIDDEN
BLKA = 512
BLKB = 32


def _leaky(x):
    return jnp.where(x >= 0, x, 0.01 * x)


def _head_kernel(lat_ref, w0, b0, w1, b1, w2, b2, wes, bes, wn, bn,
                 attr_ref, x_ref):
    h = jnp.tanh(jnp.dot(lat_ref[:], w0[:], preferred_element_type=jnp.float32) + b0[:])
    h = jnp.tanh(jnp.dot(h, w1[:], preferred_element_type=jnp.float32) + b1[:])
    h = jnp.tanh(jnp.dot(h, w2[:], preferred_element_type=jnp.float32) + b2[:])
    attr_ref[:] = jnp.dot(h, wes[:], preferred_element_type=jnp.float32) + bes[:]
    x_ref[:] = jnp.dot(h, wn[:], preferred_element_type=jnp.float32) + bn[:]


def _gnn_kernel(x_ref, attr_ref,
                wq0, wk0, wv0, we0, aweo0,
                wq1, wk1, wv1, we1, aweo1,
                g0, bb0, g1, bb1,
                featw, edgew, hsum, hexp, a128,
                xo_ref, ao_ref):
    bB = x_ref.shape[0]
    x117 = x_ref[:]
    a405 = attr_ref[:]
    x = jnp.stack([x117[:, v * HID_NODE:(v + 1) * HID_NODE] for v in range(V)],
                  axis=0).reshape(V * bB, HID_NODE)
    attr = jnp.stack([a405[:, ij * EDGE_DIM:(ij + 1) * EDGE_DIM]
                      for ij in range(V * V)], axis=0).reshape(V * V * bB, EDGE_DIM)

    convs = [(wq0, wk0, wv0, we0, aweo0), (wq1, wk1, wv1, we1, aweo1),
             (wq1, wk1, wv1, we1, aweo1)]
    norms = [(g0, bb0), (g1, bb1), None]

    for lyr in range(3):
        wq, wk, wv, we, aweo = convs[lyr]
        q = jnp.dot(x, wq[:], preferred_element_type=jnp.float32)
        k = jnp.dot(x, wk[:], preferred_element_type=jnp.float32)
        v = jnp.dot(x, wv[:], preferred_element_type=jnp.float32)
        e = jnp.dot(attr, we[:], preferred_element_type=jnp.float32)
        k4 = k.reshape(V, 1, bB, HH)
        v4 = v.reshape(V, 1, bB, HH)
        q4 = q.reshape(1, V, bB, HH)
        e4 = e.reshape(V, V, bB, HH)
        kpe = k4 + e4
        vpe = v4 + e4
        s = (q4 * kpe).reshape(V * V * bB, HH)
        score = jnp.dot(s, hsum[:], preferred_element_type=jnp.float32)
        sc4 = score.reshape(V, V, bB, HEADS)
        m = jnp.max(sc4, axis=0, keepdims=True)
        ex = jnp.exp(sc4 - m)
        den = jnp.sum(ex, axis=0, keepdims=True)
        alpha = ex / (den + 1e-16)
        ab = jnp.dot(alpha.reshape(V * V * bB, HEADS), hexp[:],
                     preferred_element_type=jnp.float32)
        contrib = ab.reshape(V, V, bB, HH) * vpe
        out = jnp.sum(contrib, axis=0).reshape(V * bB, HH)
        x_new = jnp.dot(out, a128[:], preferred_element_type=jnp.float32)
        attr_new = jnp.dot(kpe.reshape(V * V * bB, HH), aweo[:],
                           preferred_element_type=jnp.float32)
        if norms[lyr] is not None:
            g, bb = norms[lyr]
            mu = jnp.mean(x_new, axis=-1, keepdims=True)
            var = jnp.mean((x_new - mu) ** 2, axis=-1, keepdims=True)
            x_new = (x_new - mu) / jnp.sqrt(var + 1e-5) * g[:] + bb[:]
            x = _leaky(x_new)
            attr = _leaky(attr_new)
        else:
            x = x_new
            attr = attr_new

    x_out = jnp.dot(x, featw[:], preferred_element_type=jnp.float32)
    attr5 = jnp.dot(attr, edgew[:], preferred_element_type=jnp.float32)
    xj = x_out.reshape(V, bB, NODE_DIM)
    xo_ref[:] = jnp.concatenate([xj[j] for j in range(V)], axis=1)
    a5 = attr5.reshape(V * V, bB, EDGE_DIM)
    ao_ref[:] = jnp.concatenate(
        [a5[i * V + j] for i in range(V) for j in range(V) if i != j], axis=1)


def _full(shape):
    return pl.BlockSpec(shape, lambda i: tuple(0 for _ in shape))


@jax.jit
def kernel(latent_vec, params):
    B = latent_vec.shape[0]
    f32 = jnp.float32

    mlp = params["mlp"]
    w0, b0 = mlp[0]["W"], mlp[0]["b"].reshape(1, -1)
    w1, b1 = mlp[1]["W"], mlp[1]["b"].reshape(1, -1)
    w2, b2 = mlp[2]["W"], mlp[2]["b"].reshape(1, -1)

    we_ = params["edges_W"].reshape(-1, EDGE_DIM, V, V)
    we_ = 0.5 * (we_ + jnp.swapaxes(we_, 2, 3))
    wes = jnp.transpose(we_, (0, 2, 3, 1)).reshape(-1, V * V * EDGE_DIM)
    be_ = params["edges_b"].reshape(EDGE_DIM, V, V)
    be_ = 0.5 * (be_ + jnp.swapaxes(be_, 1, 2))
    bes = jnp.transpose(be_, (1, 2, 0)).reshape(1, V * V * EDGE_DIM)
    wn, bn = params["nodes_W"], params["nodes_b"].reshape(1, -1)

    hargs = [latent_vec, w0, b0, w1, b1, w2, b2, wes, bes, wn, bn]
    hspecs = [pl.BlockSpec((BLKA, latent_vec.shape[1]), lambda i: (i, 0))]
    hspecs += [_full(a.shape) for a in hargs[1:]]
    attr405, x117 = pl.pallas_call(
        _head_kernel,
        grid=(B // BLKA,),
        in_specs=hspecs,
        out_specs=[pl.BlockSpec((BLKA, V * V * EDGE_DIM), lambda i: (i, 0)),
                   pl.BlockSpec((BLKA, V * HID_NODE), lambda i: (i, 0))],
        out_shape=[jax.ShapeDtypeStruct((B, V * V * EDGE_DIM), f32),
                   jax.ShapeDtypeStruct((B, V * HID_NODE), f32)],
        compiler_params=pltpu.CompilerParams(
            dimension_semantics=("parallel",)),
    )(*hargs)

    x0 = x117
    attr0 = attr405

    a128 = (jnp.tile(jnp.eye(HIDDEN, dtype=f32), (HEADS, 1)) / HEADS)
    hsum = jnp.repeat(jnp.eye(HEADS, dtype=f32), HIDDEN, axis=0) / math.sqrt(HIDDEN)
    hexp = jnp.repeat(jnp.eye(HEADS, dtype=f32), HIDDEN, axis=1)

    conv = params["conv"]
    cargs = []
    for l in range(2):
        p = conv[l]
        cargs += [p["Wq"], p["Wk"], p["Wv"], p["We"], a128 @ p["Weo"]]
    nargs = [params["norms"][0]["g"].reshape(1, -1), params["norms"][0]["b"].reshape(1, -1),
             params["norms"][1]["g"].reshape(1, -1), params["norms"][1]["b"].reshape(1, -1)]

    gargs = [x0, attr0, *cargs, *nargs,
             params["feat_W"], params["edge_W"], hsum, hexp, a128]
    gspecs = [pl.BlockSpec((BLKB, V * HID_NODE), lambda i: (i, 0)),
              pl.BlockSpec((BLKB, V * V * EDGE_DIM), lambda i: (i, 0))]
    gspecs += [_full(a.shape) for a in gargs[2:]]

    xo, ao = pl.pallas_call(
        _gnn_kernel,
        grid=(B // BLKB,),
        in_specs=gspecs,
        out_specs=[
            pl.BlockSpec((BLKB, V * NODE_DIM), lambda i: (i, 0)),
            pl.BlockSpec((BLKB, V * (V - 1) * EDGE_DIM), lambda i: (i, 0)),
        ],
        out_shape=[
            jax.ShapeDtypeStruct((B, V * NODE_DIM), f32),
            jax.ShapeDtypeStruct((B, V * (V - 1) * EDGE_DIM), f32),
        ],
        compiler_params=pltpu.CompilerParams(
            dimension_semantics=("parallel",)),
    )(*gargs)

    x_final = xo.reshape(B * V, NODE_DIM)
    attr2 = ao.reshape(B * V * (V - 1), EDGE_DIM)

    offd = jnp.asarray([i * V + j for i in range(V) for j in range(V) if i != j],
                       dtype=jnp.int32)
    base = jnp.arange(B, dtype=jnp.int32) * V
    src2 = (base[:, None] + offd[None, :] // V).reshape(-1)
    dst2 = (base[:, None] + offd[None, :] % V).reshape(-1)
    edge_index = jnp.stack([src2, dst2], axis=0)
    batch = jnp.repeat(jnp.arange(B, dtype=jnp.int32), V)
    return x_final, edge_index, attr2, batch


# speedup vs baseline: 1.1720x; 1.1720x over previous
import math

import jax
import jax.numpy as jnp
from jax.experimental import pallas as pl
from jax.experimental.pallas import tpu as pltpu

V = 9
EDGE_DIM = 5
HID_NODE = 13
NODE_DIM = 16
HIDDEN = 32
HEADS = 4
HH = HEADS * HIDDEN
BLKA = 512
BLKB = 128


def _leaky(x):
    return jnp.where(x >= 0, x, 0.01 * x)


def _head_kernel(lat_ref, w0, b0, w1, b1, w2, b2, wes, bes, wn, bn,
                 attr_ref, x_ref):
    h = jnp.tanh(jnp.dot(lat_ref[:], w0[:], preferred_element_type=jnp.float32) + b0[:])
    h = jnp.tanh(jnp.dot(h, w1[:], preferred_element_type=jnp.float32) + b1[:])
    h = jnp.tanh(jnp.dot(h, w2[:], preferred_element_type=jnp.float32) + b2[:])
    attr_ref[:] = jnp.dot(h, wes[:], preferred_element_type=jnp.float32) + bes[:]
    x_ref[:] = jnp.dot(h, wn[:], preferred_element_type=jnp.float32) + bn[:]


def _gnn_kernel(x_ref, attr_ref,
                wq0, wk0, wv0, we0, aweo0,
                wq1, wk1, wv1, we1, aweo1,
                g0, bb0, g1, bb1,
                featw, edgew, hsum, hexp, a128,
                xo_ref, ao_ref):
    bB = x_ref.shape[0]
    x117 = x_ref[:]
    a405 = attr_ref[:]
    x = jnp.stack([x117[:, v * HID_NODE:(v + 1) * HID_NODE] for v in range(V)],
                  axis=0).reshape(V * bB, HID_NODE)
    attr = jnp.stack([a405[:, ij * EDGE_DIM:(ij + 1) * EDGE_DIM]
                      for ij in range(V * V)], axis=0).reshape(V * V * bB, EDGE_DIM)

    convs = [(wq0, wk0, wv0, we0, aweo0), (wq1, wk1, wv1, we1, aweo1),
             (wq1, wk1, wv1, we1, aweo1)]
    norms = [(g0, bb0), (g1, bb1), None]

    for lyr in range(3):
        wq, wk, wv, we, aweo = convs[lyr]
        q = jnp.dot(x, wq[:], preferred_element_type=jnp.float32)
        k = jnp.dot(x, wk[:], preferred_element_type=jnp.float32)
        v = jnp.dot(x, wv[:], preferred_element_type=jnp.float32)
        e = jnp.dot(attr, we[:], preferred_element_type=jnp.float32)
        k4 = k.reshape(V, 1, bB, HH)
        v4 = v.reshape(V, 1, bB, HH)
        q4 = q.reshape(1, V, bB, HH)
        e4 = e.reshape(V, V, bB, HH)
        kpe = k4 + e4
        vpe = v4 + e4
        s = (q4 * kpe).reshape(V * V * bB, HH)
        score = jnp.dot(s, hsum[:], preferred_element_type=jnp.float32)
        sc4 = score.reshape(V, V, bB, HEADS)
        m = jnp.max(sc4, axis=0, keepdims=True)
        ex = jnp.exp(sc4 - m)
        den = jnp.sum(ex, axis=0, keepdims=True)
        alpha = ex / (den + 1e-16)
        ab = jnp.dot(alpha.reshape(V * V * bB, HEADS), hexp[:],
                     preferred_element_type=jnp.float32)
        contrib = ab.reshape(V, V, bB, HH) * vpe
        out = jnp.sum(contrib, axis=0).reshape(V * bB, HH)
        x_new = jnp.dot(out, a128[:], preferred_element_type=jnp.float32)
        attr_new = jnp.dot(kpe.reshape(V * V * bB, HH), aweo[:],
                           preferred_element_type=jnp.float32)
        if norms[lyr] is not None:
            g, bb = norms[lyr]
            mu = jnp.mean(x_new, axis=-1, keepdims=True)
            var = jnp.mean((x_new - mu) ** 2, axis=-1, keepdims=True)
            x_new = (x_new - mu) / jnp.sqrt(var + 1e-5) * g[:] + bb[:]
            x = _leaky(x_new)
            attr = _leaky(attr_new)
        else:
            x = x_new
            attr = attr_new

    x_out = jnp.dot(x, featw[:], preferred_element_type=jnp.float32)
    attr5 = jnp.dot(attr, edgew[:], preferred_element_type=jnp.float32)
    xj = x_out.reshape(V, bB, NODE_DIM)
    xo_ref[:] = jnp.concatenate([xj[j] for j in range(V)], axis=1)
    a5 = attr5.reshape(V * V, bB, EDGE_DIM)
    ao_ref[:] = jnp.concatenate(
        [a5[i * V + j] for i in range(V) for j in range(V) if i != j], axis=1)


def _full(shape):
    return pl.BlockSpec(shape, lambda i: tuple(0 for _ in shape))


@jax.jit
def kernel(latent_vec, params):
    B = latent_vec.shape[0]
    f32 = jnp.float32

    mlp = params["mlp"]
    w0, b0 = mlp[0]["W"], mlp[0]["b"].reshape(1, -1)
    w1, b1 = mlp[1]["W"], mlp[1]["b"].reshape(1, -1)
    w2, b2 = mlp[2]["W"], mlp[2]["b"].reshape(1, -1)

    we_ = params["edges_W"].reshape(-1, EDGE_DIM, V, V)
    we_ = 0.5 * (we_ + jnp.swapaxes(we_, 2, 3))
    wes = jnp.transpose(we_, (0, 2, 3, 1)).reshape(-1, V * V * EDGE_DIM)
    be_ = params["edges_b"].reshape(EDGE_DIM, V, V)
    be_ = 0.5 * (be_ + jnp.swapaxes(be_, 1, 2))
    bes = jnp.transpose(be_, (1, 2, 0)).reshape(1, V * V * EDGE_DIM)
    wn, bn = params["nodes_W"], params["nodes_b"].reshape(1, -1)

    hargs = [latent_vec, w0, b0, w1, b1, w2, b2, wes, bes, wn, bn]
    hspecs = [pl.BlockSpec((BLKA, latent_vec.shape[1]), lambda i: (i, 0))]
    hspecs += [_full(a.shape) for a in hargs[1:]]
    attr405, x117 = pl.pallas_call(
        _head_kernel,
        grid=(B // BLKA,),
        in_specs=hspecs,
        out_specs=[pl.BlockSpec((BLKA, V * V * EDGE_DIM), lambda i: (i, 0)),
                   pl.BlockSpec((BLKA, V * HID_NODE), lambda i: (i, 0))],
        out_shape=[jax.ShapeDtypeStruct((B, V * V * EDGE_DIM), f32),
                   jax.ShapeDtypeStruct((B, V * HID_NODE), f32)],
        compiler_params=pltpu.CompilerParams(
            dimension_semantics=("parallel",)),
    )(*hargs)

    x0 = x117
    attr0 = attr405

    a128 = (jnp.tile(jnp.eye(HIDDEN, dtype=f32), (HEADS, 1)) / HEADS)
    hsum = jnp.repeat(jnp.eye(HEADS, dtype=f32), HIDDEN, axis=0) / math.sqrt(HIDDEN)
    hexp = jnp.repeat(jnp.eye(HEADS, dtype=f32), HIDDEN, axis=1)

    conv = params["conv"]
    cargs = []
    for l in range(2):
        p = conv[l]
        cargs += [p["Wq"], p["Wk"], p["Wv"], p["We"], a128 @ p["Weo"]]
    nargs = [params["norms"][0]["g"].reshape(1, -1), params["norms"][0]["b"].reshape(1, -1),
             params["norms"][1]["g"].reshape(1, -1), params["norms"][1]["b"].reshape(1, -1)]

    gargs = [x0, attr0, *cargs, *nargs,
             params["feat_W"], params["edge_W"], hsum, hexp, a128]
    gspecs = [pl.BlockSpec((BLKB, V * HID_NODE), lambda i: (i, 0)),
              pl.BlockSpec((BLKB, V * V * EDGE_DIM), lambda i: (i, 0))]
    gspecs += [_full(a.shape) for a in gargs[2:]]

    xo, ao = pl.pallas_call(
        _gnn_kernel,
        grid=(B // BLKB,),
        in_specs=gspecs,
        out_specs=[
            pl.BlockSpec((BLKB, V * NODE_DIM), lambda i: (i, 0)),
            pl.BlockSpec((BLKB, V * (V - 1) * EDGE_DIM), lambda i: (i, 0)),
        ],
        out_shape=[
            jax.ShapeDtypeStruct((B, V * NODE_DIM), f32),
            jax.ShapeDtypeStruct((B, V * (V - 1) * EDGE_DIM), f32),
        ],
        compiler_params=pltpu.CompilerParams(
            dimension_semantics=("parallel",)),
    )(*gargs)

    x_final = xo.reshape(B * V, NODE_DIM)
    attr2 = ao.reshape(B * V * (V - 1), EDGE_DIM)

    offd = jnp.asarray([i * V + j for i in range(V) for j in range(V) if i != j],
                       dtype=jnp.int32)
    base = jnp.arange(B, dtype=jnp.int32) * V
    src2 = (base[:, None] + offd[None, :] // V).reshape(-1)
    dst2 = (base[:, None] + offd[None, :] % V).reshape(-1)
    edge_index = jnp.stack([src2, dst2], axis=0)
    batch = jnp.repeat(jnp.arange(B, dtype=jnp.int32), V)
    return x_final, edge_index, attr2, batch
